# baseline (speedup 1.0000x reference)
import functools

import numpy as np

import jax
import jax.numpy as jnp
from jax import lax
from jax.experimental import pallas as pl
from jax.experimental.pallas import tpu as pltpu
from jax.experimental.pallas import tpu_sc as plsc

N = 10000
E = 320000
D = 128

NC = 2
NS = 16
NW = NC * NS
C = 128
WB = 128
N_PAD = 10240
SLAB = N_PAD // NS
E_PAD = 327680
CH = E_PAD // (NW * C)
TCH = E_PAD // C
R = 1024
GRID = N_PAD // R

EC = 64
ETCH = E_PAD // EC
ECPS = ETCH // NW
ESTG = 40
ENB = 4

_mesh = plsc.VectorSubcoreMesh(core_axis_name="c", subcore_axis_name="s")


@functools.partial(
    pl.kernel,
    mesh=_mesh,
    out_type=jax.ShapeDtypeStruct((NC, N_PAD), jnp.float32),
    scratch_types=[
        pltpu.VMEM((CH, C), jnp.int32),
        pltpu.VMEM((C,), jnp.float32),
        pltpu.VMEM_SHARED((N_PAD,), jnp.float32),
    ],
)
def _deg_kernel(cidx_hbm, ones_hbm, zeros_hbm, deg_out, cidx_v, ones_v, acc):
    cid = lax.axis_index("c")
    sid = lax.axis_index("s")
    wid = sid * NC + cid

    pltpu.sync_copy(cidx_hbm.at[pl.ds(wid * CH, CH)], cidx_v)
    pltpu.sync_copy(ones_hbm, ones_v)
    pltpu.sync_copy(zeros_hbm, acc.at[pl.ds(sid * SLAB, SLAB)])
    plsc.subcore_barrier()

    @pl.loop(0, CH)
    def _(j):
        pltpu.sync_copy(ones_v, acc.at[cidx_v.at[j]], add=True)

    plsc.subcore_barrier()
    pltpu.sync_copy(
        acc.at[pl.ds(sid * SLAB, SLAB)],
        deg_out.at[cid, pl.ds(sid * SLAB, SLAB)],
    )


@functools.partial(
    pl.kernel,
    mesh=_mesh,
    out_type=[jax.ShapeDtypeStruct((N_PAD, D), jnp.float32),
              jax.ShapeDtypeStruct((N_PAD, D), jnp.float32)],
    scratch_types=[
        pltpu.VMEM((ESTG, EC), jnp.int32),
        pltpu.VMEM((ESTG, EC), jnp.int32),
        pltpu.VMEM_SHARED((N_PAD, D), jnp.float32),
        [pltpu.SemaphoreType.DMA] * ENB,
        [pltpu.SemaphoreType.DMA] * ENB,
    ],
)
def _edge_kernel(ridx_hbm, cidx_hbm, h_hbm, zrows_hbm, out0_hbm, out1_hbm,
                 ridx_v, cidx_v, acc, gsems, ssems):
    cid = lax.axis_index("c")
    sid = lax.axis_index("s")

    pltpu.sync_copy(zrows_hbm, acc.at[pl.ds(sid * SLAB, SLAB)])
    plsc.subcore_barrier()

    def body(msg_v):
        chunk0 = (cid * NS + sid) * ECPS
        for stage in range(ECPS // ESTG):
            base = chunk0 + stage * ESTG
            pltpu.sync_copy(ridx_hbm.at[pl.ds(base, ESTG)], ridx_v)
            pltpu.sync_copy(cidx_hbm.at[pl.ds(base, ESTG)], cidx_v)
            for b in range(ENB):
                pltpu.async_copy(h_hbm.at[ridx_v.at[b]], msg_v.at[b],
                                 gsems[b])

            @pl.loop(0, ESTG, step=ENB)
            def _(j):
                for b in range(ENB):
                    pltpu.make_async_copy(
                        h_hbm.at[ridx_v.at[0]], msg_v.at[b],
                        gsems[b]).wait()
                    pltpu.async_copy(msg_v.at[b], acc.at[cidx_v.at[j + b]],
                                     ssems[b])
                for b in range(ENB):
                    nxt = j + b + ENB

                    @pl.when(nxt < ESTG)
                    def _():
                        pltpu.make_async_copy(
                            msg_v.at[b], acc.at[cidx_v.at[0]],
                            ssems[b]).wait()
                        pltpu.async_copy(
                            h_hbm.at[ridx_v.at[nxt]], msg_v.at[b], gsems[b])

            for b in range(ENB):
                pltpu.make_async_copy(
                    msg_v.at[b], acc.at[cidx_v.at[0]], ssems[b]).wait()

    pl.run_scoped(body, pltpu.VMEM((ENB, EC, D), jnp.float32))
    plsc.subcore_barrier()

    @pl.loop(0, SLAB // WB)
    def _(k):
        base = sid * SLAB + k * WB

        @pl.when(cid == 0)
        def _():
            pltpu.sync_copy(acc.at[pl.ds(base, WB)],
                            out0_hbm.at[pl.ds(base, WB)])

        @pl.when(cid == 1)
        def _():
            pltpu.sync_copy(acc.at[pl.ds(base, WB)],
                            out1_hbm.at[pl.ds(base, WB)])


def _dinv_block(da, db):
    d = da + db
    return jnp.where(d > 0, lax.rsqrt(jnp.where(d > 0, d, 1.0)), 0.0)


def _tc_mm_body(x_ref, w_ref, o_ref):
    o_ref[...] = jnp.dot(x_ref[...], w_ref[...],
                         preferred_element_type=jnp.float32)


def _tc_scale_body(h_ref, da_ref, db_ref, o_ref):
    o_ref[...] = h_ref[...] * _dinv_block(da_ref[...], db_ref[...])


def _tc_mid_body(pa_ref, pb_ref, da_ref, db_ref, b_ref, w_ref, o_ref):
    dinv = _dinv_block(da_ref[...], db_ref[...])
    y = (pa_ref[...] + pb_ref[...]) * dinv + b_ref[...]
    h = jnp.dot(y, w_ref[...], preferred_element_type=jnp.float32)
    o_ref[...] = h * dinv


def _tc_out_body(pa_ref, pb_ref, da_ref, db_ref, b_ref, o_ref):
    dinv = _dinv_block(da_ref[...], db_ref[...])
    o_ref[...] = (pa_ref[...] + pb_ref[...]) * dinv + b_ref[...]


_row_spec = pl.BlockSpec((R, D), lambda i: (i, 0))
_n_shape = jax.ShapeDtypeStruct((N, D), jnp.float32)
_deg_spec = pl.BlockSpec((R, 1), lambda i: (i, 0))
_w_spec = pl.BlockSpec((D, D), lambda i: (0, 0))
_b_spec = pl.BlockSpec((1, D), lambda i: (0, 0))
_out_shape = jax.ShapeDtypeStruct((N_PAD, D), jnp.float32)

_tc_mm = pl.pallas_call(
    _tc_mm_body, grid=(GRID,),
    in_specs=[_row_spec, _w_spec],
    out_specs=_row_spec, out_shape=_n_shape)

_tc_scale = pl.pallas_call(
    _tc_scale_body, grid=(GRID,),
    in_specs=[_row_spec, _deg_spec, _deg_spec],
    out_specs=_row_spec, out_shape=_n_shape)

_tc_mid = pl.pallas_call(
    _tc_mid_body, grid=(GRID,),
    in_specs=[_row_spec, _row_spec, _deg_spec, _deg_spec, _b_spec, _w_spec],
    out_specs=_row_spec, out_shape=_n_shape)

_tc_out = pl.pallas_call(
    _tc_out_body, grid=(GRID,),
    in_specs=[_row_spec, _row_spec, _deg_spec, _deg_spec, _b_spec],
    out_specs=_row_spec, out_shape=_n_shape)


_PAD_ROW = np.asarray(np.arange(E_PAD - E) % N, dtype=np.int32)
_PAD_COL = np.asarray(N + np.arange(E_PAD - E) % (N_PAD - N), dtype=np.int32)
_ONES_C = np.ones((C,), np.float32)
_ZEROS_SLAB = np.zeros((SLAB,), np.float32)
_ZEROS_ROWS = np.zeros((SLAB, D), np.float32)


def kernel(x, edge_index, W1, b1, W2, b2):
    row = edge_index[0]
    col = edge_index[1]
    row_p = jnp.concatenate([row, _PAD_ROW])
    col_p = jnp.concatenate([col, _PAD_COL])
    ridx = row_p.reshape(ETCH, EC)
    cidx = col_p.reshape(ETCH, EC)
    cidx_deg = col_p.reshape(TCH, C)
    b1r = b1.reshape(1, D)
    b2r = b2.reshape(1, D)

    deg_parts = _deg_kernel(cidx_deg, _ONES_C, _ZEROS_SLAB)
    da = deg_parts[0].reshape(N_PAD, 1)
    db = deg_parts[1].reshape(N_PAD, 1)

    hm = _tc_mm(x, W1)
    h1 = _tc_scale(hm, da, db)
    p1a, p1b = _edge_kernel(ridx, cidx, h1, _ZEROS_ROWS)
    h2 = _tc_mid(p1a, p1b, da, db, b1r, W2)
    p2a, p2b = _edge_kernel(ridx, cidx, h2, _ZEROS_ROWS)
    return _tc_out(p2a, p2b, da, db, b2r)


# speedup vs baseline: 1.0198x; 1.0198x over previous
import functools

import numpy as np

import jax
import jax.numpy as jnp
from jax import lax
from jax.experimental import pallas as pl
from jax.experimental.pallas import tpu as pltpu
from jax.experimental.pallas import tpu_sc as plsc

N = 10000
E = 320000
D = 128

NC = 2
NS = 16
NW = NC * NS
C = 128
WB = 128
N_PAD = 10240
SLAB = N_PAD // NS
E_PAD = 327680
CH = E_PAD // (NW * C)
TCH = E_PAD // C
R = 1024
GRID = N_PAD // R

CF = 80
CS = TCH // NS - CF
FAST_CID = 0
STAGES_F = (40, 40)
STAGES_S = (40, 40)

_mesh = plsc.VectorSubcoreMesh(core_axis_name="c", subcore_axis_name="s")


@functools.partial(
    pl.kernel,
    mesh=_mesh,
    out_type=jax.ShapeDtypeStruct((NC, N_PAD), jnp.float32),
    scratch_types=[
        pltpu.VMEM((CH, C), jnp.int32),
        pltpu.VMEM((C,), jnp.float32),
        pltpu.VMEM_SHARED((N_PAD,), jnp.float32),
    ],
)
def _deg_kernel(cidx_hbm, ones_hbm, zeros_hbm, deg_out, cidx_v, ones_v, acc):
    cid = lax.axis_index("c")
    sid = lax.axis_index("s")
    wid = sid * NC + cid

    pltpu.sync_copy(cidx_hbm.at[pl.ds(wid * CH, CH)], cidx_v)
    pltpu.sync_copy(ones_hbm, ones_v)
    pltpu.sync_copy(zeros_hbm, acc.at[pl.ds(sid * SLAB, SLAB)])
    plsc.subcore_barrier()

    @pl.loop(0, CH)
    def _(j):
        pltpu.sync_copy(ones_v, acc.at[cidx_v.at[j]], add=True)

    plsc.subcore_barrier()
    pltpu.sync_copy(
        acc.at[pl.ds(sid * SLAB, SLAB)],
        deg_out.at[cid, pl.ds(sid * SLAB, SLAB)],
    )


@functools.partial(
    pl.kernel,
    mesh=_mesh,
    out_type=[jax.ShapeDtypeStruct((N_PAD, D), jnp.float32),
              jax.ShapeDtypeStruct((N_PAD, D), jnp.float32)],
    scratch_types=[
        pltpu.VMEM((40, C), jnp.int32),
        pltpu.VMEM((40, C), jnp.int32),
        pltpu.VMEM_SHARED((N_PAD, D), jnp.float32),
        pltpu.SemaphoreType.DMA,
        pltpu.SemaphoreType.DMA,
    ],
)
def _edge_kernel(ridx_hbm, cidx_hbm, h_hbm, zrows_hbm, out0_hbm, out1_hbm,
                 ridx_v, cidx_v, acc, sem0, sem1):
    cid = lax.axis_index("c")
    sid = lax.axis_index("s")
    sems = (sem0, sem1)
    nb = 2

    pltpu.sync_copy(zrows_hbm, acc.at[pl.ds(sid * SLAB, SLAB)])
    plsc.subcore_barrier()

    def pipeline(msg_v, chunk0, stages):
        off = 0
        for sz in stages:
            base = chunk0 + off
            off += sz
            pltpu.sync_copy(ridx_hbm.at[pl.ds(base, sz)],
                            ridx_v.at[pl.ds(0, sz)])
            pltpu.sync_copy(cidx_hbm.at[pl.ds(base, sz)],
                            cidx_v.at[pl.ds(0, sz)])
            for b in range(nb):
                pltpu.async_copy(h_hbm.at[ridx_v.at[b]], msg_v.at[b],
                                 sems[b])

            @pl.loop(0, sz, step=nb)
            def _(j):
                for b in range(nb):
                    pltpu.make_async_copy(
                        h_hbm.at[ridx_v.at[0]], msg_v.at[b], sems[b]).wait()
                    pltpu.sync_copy(msg_v.at[b], acc.at[cidx_v.at[j + b]],
                                    add=True)
                    nxt = j + b + nb

                    @pl.when(nxt < sz)
                    def _():
                        pltpu.async_copy(
                            h_hbm.at[ridx_v.at[nxt]], msg_v.at[b], sems[b])

    def body(msg_v):
        @pl.when(cid == FAST_CID)
        def _():
            pipeline(msg_v, sid * CF, STAGES_F)

        @pl.when(cid != FAST_CID)
        def _():
            pipeline(msg_v, NS * CF + sid * CS, STAGES_S)

    pl.run_scoped(body, pltpu.VMEM((2, C, D), jnp.float32))
    plsc.subcore_barrier()

    @pl.loop(0, SLAB // WB)
    def _(k):
        base = sid * SLAB + k * WB

        @pl.when(cid == 0)
        def _():
            pltpu.sync_copy(acc.at[pl.ds(base, WB)],
                            out0_hbm.at[pl.ds(base, WB)])

        @pl.when(cid == 1)
        def _():
            pltpu.sync_copy(acc.at[pl.ds(base, WB)],
                            out1_hbm.at[pl.ds(base, WB)])


def _dinv_block(da, db):
    d = da + db
    return jnp.where(d > 0, lax.rsqrt(jnp.where(d > 0, d, 1.0)), 0.0)


def _tc_mm_body(x_ref, w_ref, o_ref):
    o_ref[...] = jnp.dot(x_ref[...], w_ref[...],
                         preferred_element_type=jnp.float32)


def _tc_scale_body(h_ref, da_ref, db_ref, o_ref):
    o_ref[...] = h_ref[...] * _dinv_block(da_ref[...], db_ref[...])


def _tc_mid_body(pa_ref, pb_ref, da_ref, db_ref, b_ref, w_ref, o_ref):
    dinv = _dinv_block(da_ref[...], db_ref[...])
    y = (pa_ref[...] + pb_ref[...]) * dinv + b_ref[...]
    h = jnp.dot(y, w_ref[...], preferred_element_type=jnp.float32)
    o_ref[...] = h * dinv


def _tc_out_body(pa_ref, pb_ref, da_ref, db_ref, b_ref, o_ref):
    dinv = _dinv_block(da_ref[...], db_ref[...])
    o_ref[...] = (pa_ref[...] + pb_ref[...]) * dinv + b_ref[...]


_row_spec = pl.BlockSpec((R, D), lambda i: (i, 0))
_n_shape = jax.ShapeDtypeStruct((N, D), jnp.float32)
_deg_spec = pl.BlockSpec((R, 1), lambda i: (i, 0))
_w_spec = pl.BlockSpec((D, D), lambda i: (0, 0))
_b_spec = pl.BlockSpec((1, D), lambda i: (0, 0))
_out_shape = jax.ShapeDtypeStruct((N_PAD, D), jnp.float32)

_tc_mm = pl.pallas_call(
    _tc_mm_body, grid=(GRID,),
    in_specs=[_row_spec, _w_spec],
    out_specs=_row_spec, out_shape=_n_shape)

_tc_scale = pl.pallas_call(
    _tc_scale_body, grid=(GRID,),
    in_specs=[_row_spec, _deg_spec, _deg_spec],
    out_specs=_row_spec, out_shape=_n_shape)

_tc_mid = pl.pallas_call(
    _tc_mid_body, grid=(GRID,),
    in_specs=[_row_spec, _row_spec, _deg_spec, _deg_spec, _b_spec, _w_spec],
    out_specs=_row_spec, out_shape=_n_shape)

_tc_out = pl.pallas_call(
    _tc_out_body, grid=(GRID,),
    in_specs=[_row_spec, _row_spec, _deg_spec, _deg_spec, _b_spec],
    out_specs=_row_spec, out_shape=_n_shape)


_PAD_ROW = np.asarray(np.arange(E_PAD - E) % N, dtype=np.int32)
_PAD_COL = np.asarray(N + np.arange(E_PAD - E) % (N_PAD - N), dtype=np.int32)
_ONES_C = np.ones((C,), np.float32)
_ZEROS_SLAB = np.zeros((SLAB,), np.float32)
_ZEROS_ROWS = np.zeros((SLAB, D), np.float32)


def kernel(x, edge_index, W1, b1, W2, b2):
    row = edge_index[0]
    col = edge_index[1]
    row_p = jnp.concatenate([row, _PAD_ROW])
    col_p = jnp.concatenate([col, _PAD_COL])
    ridx = row_p.reshape(TCH, C)
    cidx = col_p.reshape(TCH, C)
    b1r = b1.reshape(1, D)
    b2r = b2.reshape(1, D)

    deg_parts = _deg_kernel(cidx, _ONES_C, _ZEROS_SLAB)
    da = deg_parts[0].reshape(N_PAD, 1)
    db = deg_parts[1].reshape(N_PAD, 1)

    hm = _tc_mm(x, W1)
    h1 = _tc_scale(hm, da, db)
    p1a, p1b = _edge_kernel(ridx, cidx, h1, _ZEROS_ROWS)
    h2 = _tc_mid(p1a, p1b, da, db, b1r, W2)
    p2a, p2b = _edge_kernel(ridx, cidx, h2, _ZEROS_ROWS)
    return _tc_out(p2a, p2b, da, db, b2r)
